# baseline (speedup 1.0000x reference)
import jax
import jax.numpy as jnp
import numpy as np
from jax.experimental import pallas as pl
from jax.experimental.pallas import tpu as pltpu

_EPS = 0.1
_NSLOT = 4


def _fill_kernel(x_hbm, o_hbm, tmpl, sems, xsem):
    i = pl.program_id(0)
    n = pl.num_programs(0)
    p, c, h, w = tmpl.shape

    @pl.when(i == 0)
    def _init():
        g = jax.lax.broadcasted_iota(jnp.int32, (p, c, h, w), 0)
        ci = jax.lax.broadcasted_iota(jnp.int32, (p, c, h, w), 1)
        hi = jax.lax.broadcasted_iota(jnp.int32, (p, c, h, w), 2)
        wi = jax.lax.broadcasted_iota(jnp.int32, (p, c, h, w), 3)
        flat = (ci * h + hi) * w + wi
        tmpl[...] = jnp.where(g == flat, _EPS, 0.0).astype(tmpl.dtype)
        pltpu.make_async_copy(x_hbm, o_hbm.at[:, pl.ds(0, 1)], xsem).start()

    slot = jax.lax.rem(i, _NSLOT)

    @pl.when(i >= _NSLOT)
    def _wait_prev():
        pltpu.make_async_copy(
            tmpl, o_hbm.at[i - _NSLOT, pl.ds(1, p)], sems.at[slot]
        ).wait()

    pltpu.make_async_copy(tmpl, o_hbm.at[i, pl.ds(1, p)], sems.at[slot]).start()

    @pl.when(i == n - 1)
    def _drain():
        for j in range(_NSLOT):
            it = n - _NSLOT + j
            pltpu.make_async_copy(
                tmpl, o_hbm.at[it, pl.ds(1, p)], sems.at[it % _NSLOT]
            ).wait()
        pltpu.make_async_copy(x_hbm, o_hbm.at[:, pl.ds(0, 1)], xsem).wait()


def kernel(x):
    n = x.shape[0]
    inner = x.shape[1:]
    p = int(np.prod(inner))
    xf = x.reshape((n, 1) + tuple(inner))
    return pl.pallas_call(
        _fill_kernel,
        grid=(n,),
        in_specs=[pl.BlockSpec(memory_space=pl.ANY)],
        out_specs=pl.BlockSpec(memory_space=pl.ANY),
        out_shape=jax.ShapeDtypeStruct((n, 1 + p) + tuple(inner), x.dtype),
        scratch_shapes=[
            pltpu.VMEM((p,) + tuple(inner), x.dtype),
            pltpu.SemaphoreType.DMA((_NSLOT,)),
            pltpu.SemaphoreType.DMA,
        ],
        compiler_params=pltpu.CompilerParams(
            dimension_semantics=("arbitrary",),
        ),
    )(xf)


# speedup vs baseline: 1.2447x; 1.2447x over previous
import functools

import jax
import jax.numpy as jnp
import numpy as np
from jax import lax
from jax.experimental import pallas as pl
from jax.experimental.pallas import tpu as pltpu
from jax.experimental.pallas import tpu_sc as plsc

_EPS = 0.1
_CH = 56


def _to_z_sc(n, p, x_hbm, o_hbm, bufs, xbuf, sems):
    info = plsc.get_sparse_core_info()
    nc, ns = info.num_cores, info.num_subcores
    nw = nc * ns
    nch = p // _CH
    per_w = n // nw

    wid = lax.axis_index("s") * nc + lax.axis_index("c")
    lanes = jnp.arange(16, dtype=jnp.int32)
    zeros16 = jnp.zeros((16,), jnp.float32)
    eps16 = jnp.full((16,), _EPS, jnp.float32)

    def _zero_row(r, _):
        def _zero_seg(q, _):
            for b in range(2):
                bufs[b, r, pl.ds(q * 16, 16)] = zeros16
            return 0

        return lax.fori_loop(0, p // 16, _zero_seg, 0)

    lax.fori_loop(0, _CH, _zero_row, 0)

    def _chunk_dmas(b, c, start):
        for s in range(per_w):
            batch = wid * per_w + s
            cp = pltpu.make_async_copy(
                bufs.at[b],
                o_hbm.at[batch, pl.ds(1 + c * _CH, _CH), :],
                sems.at[b],
            )
            if start:
                cp.start()
            else:
                cp.wait()

    def _pair(t, _):
        for b in range(2):
            c = 2 * t + b

            @pl.when(t > 0)
            def _wait_prev():
                _chunk_dmas(b, c - 2, start=False)

            for k in range(-(-_CH // 16)):
                j = lanes + k * 16
                row_ok = j < _CH
                col_new = c * _CH + j
                col_old = col_new - 2 * _CH
                plsc.store_scatter(
                    bufs.at[b], [j, col_old], zeros16, mask=row_ok & (col_old >= 0)
                )
                plsc.store_scatter(bufs.at[b], [j, col_new], eps16, mask=row_ok)
            _chunk_dmas(b, c, start=True)
        return 0

    lax.fori_loop(0, nch // 2, _pair, 0)

    for s in range(per_w):
        batch = wid * per_w + s
        pltpu.sync_copy(x_hbm.at[batch], xbuf)
        pltpu.sync_copy(xbuf, o_hbm.at[batch, 0])

    for b in range(2):
        _chunk_dmas(b, nch - 2 + b, start=False)


def kernel(x):
    n = x.shape[0]
    inner = x.shape[1:]
    p = int(np.prod(inner))
    xf = x.reshape(n, p)
    mesh = plsc.VectorSubcoreMesh(core_axis_name="c", subcore_axis_name="s")
    out = pl.kernel(
        functools.partial(_to_z_sc, n, p),
        out_type=jax.ShapeDtypeStruct((n, 1 + p, p), x.dtype),
        mesh=mesh,
        scratch_types=[
            pltpu.VMEM((2, _CH, p), jnp.float32),
            pltpu.VMEM((p,), jnp.float32),
            pltpu.SemaphoreType.DMA((2,)),
        ],
        compiler_params=pltpu.CompilerParams(
            use_tc_tiling_on_sc=False, needs_layout_passes=False
        ),
    )(xf)
    return out.reshape((n, 1 + p) + tuple(inner))


# speedup vs baseline: 2.3618x; 1.8974x over previous
import functools

import jax
import jax.numpy as jnp
import numpy as np
from jax import lax
from jax.experimental import pallas as pl
from jax.experimental.pallas import tpu as pltpu
from jax.experimental.pallas import tpu_sc as plsc

_EPS = 0.1
_CH = 56


def _to_z_sc(n, p, x_hbm, o_hbm, bufs, tail, sems, tsem):
    info = plsc.get_sparse_core_info()
    nc, ns = info.num_cores, info.num_subcores
    nw = nc * ns
    nch = (1 + p) // _CH
    per_w = n // nw

    wid = lax.axis_index("s") * nc + lax.axis_index("c")
    lanes = jnp.arange(16, dtype=jnp.int32)
    zeros16 = jnp.zeros((16,), jnp.float32)
    eps16 = jnp.full((16,), _EPS, jnp.float32)

    def _zero_row(r, _):
        def _zero_seg(q, _):
            for b in range(2):
                bufs[b, r, pl.ds(q * 16, 16)] = zeros16
            return 0

        return lax.fori_loop(0, p // 16, _zero_seg, 0)

    lax.fori_loop(0, _CH, _zero_row, 0)

    def _zero_tail(q, _):
        tail[0, pl.ds(q * 16, 16)] = zeros16
        return 0

    lax.fori_loop(0, p // 16, _zero_tail, 0)
    plsc.store_scatter(
        tail,
        [jnp.zeros((16,), jnp.int32), jnp.full((16,), p - 1, jnp.int32)],
        eps16,
        mask=lanes == 0,
    )

    def _chunk_dmas(b, c, start):
        for s in range(per_w):
            batch = wid * per_w + s
            cp = pltpu.make_async_copy(
                bufs.at[b],
                o_hbm.at[batch, pl.ds(c * _CH, _CH), :],
                sems.at[b],
            )
            if start:
                cp.start()
            else:
                cp.wait()

    for k in range(-(-_CH // 16)):
        j = lanes + k * 16
        row_ok = (j >= 1) & (j < _CH)
        plsc.store_scatter(bufs.at[0], [j, j - 1], eps16, mask=row_ok)
    for s in range(per_w):
        batch = wid * per_w + s
        pltpu.sync_copy(x_hbm.at[batch], bufs.at[0, 0])
        pltpu.make_async_copy(
            bufs.at[0], o_hbm.at[batch, pl.ds(0, _CH), :], sems.at[0]
        ).start()
        pltpu.make_async_copy(
            bufs.at[0], o_hbm.at[batch, pl.ds(0, _CH), :], sems.at[0]
        ).wait()
        pltpu.make_async_copy(
            tail, o_hbm.at[batch, pl.ds(nch * _CH, 1), :], tsem
        ).start()

    def _pair(t, _):
        for b in range(2):
            c = 2 * t + b + 1

            @pl.when(c > 2)
            def _wait_prev():
                _chunk_dmas(b, c - 2, start=False)

            for k in range(-(-_CH // 16)):
                j = lanes + k * 16
                row_ok = j < _CH
                col_new = c * _CH + j - 1
                col_old = col_new - 2 * _CH
                plsc.store_scatter(
                    bufs.at[b],
                    [j, col_old],
                    zeros16,
                    mask=row_ok & (col_old >= 0) & (col_old < p),
                )
                plsc.store_scatter(
                    bufs.at[b], [j, col_new], eps16, mask=row_ok & (col_new < p)
                )
            _chunk_dmas(b, c, start=True)
        return 0

    for k in range(-(-_CH // 16)):
        j = lanes + k * 16
        row_ok = (j >= 1) & (j < _CH)
        plsc.store_scatter(bufs.at[0], [j, j - 1], zeros16, mask=row_ok)

    lax.fori_loop(0, (nch - 1) // 2, _pair, 0)

    _chunk_dmas(1, nch - 2, start=False)
    _chunk_dmas(0, nch - 1, start=False)
    for s in range(per_w):
        batch = wid * per_w + s
        pltpu.make_async_copy(
            tail, o_hbm.at[batch, pl.ds(nch * _CH, 1), :], tsem
        ).wait()


def kernel(x):
    n = x.shape[0]
    inner = x.shape[1:]
    p = int(np.prod(inner))
    xf = x.reshape(n, p)
    mesh = plsc.VectorSubcoreMesh(core_axis_name="c", subcore_axis_name="s")
    out = pl.kernel(
        functools.partial(_to_z_sc, n, p),
        out_type=jax.ShapeDtypeStruct((n, 1 + p, p), x.dtype),
        mesh=mesh,
        scratch_types=[
            pltpu.VMEM((2, _CH, p), jnp.float32),
            pltpu.VMEM((1, p), jnp.float32),
            pltpu.SemaphoreType.DMA((2,)),
            pltpu.SemaphoreType.DMA,
        ],
        compiler_params=pltpu.CompilerParams(
            use_tc_tiling_on_sc=True, needs_layout_passes=False
        ),
    )(xf)
    return out.reshape((n, 1 + p) + tuple(inner))
